# baseline (speedup 1.0000x reference)
import jax
import jax.numpy as jnp
from jax.experimental import pallas as pl


def _body(x_ref, nx_ref, wl_ref, wr_ref, o_ref):
    k = nx_ref.shape[1]
    neigh = jnp.sum(nx_ref[...], axis=1) * (1.0 / k)
    o_ref[...] = (
        jnp.dot(x_ref[...], wl_ref[...], preferred_element_type=jnp.float32)
        + jnp.dot(neigh, wr_ref[...], preferred_element_type=jnp.float32)
    )


def kernel(x, neigh_x, W_l, W_r):
    n, d_in = x.shape
    _, k, _ = neigh_x.shape
    d_out = W_l.shape[0]
    bn = 200
    assert n % bn == 0
    wl_t = W_l.T
    wr_t = W_r.T
    return pl.pallas_call(
        _body,
        grid=(n // bn,),
        in_specs=[
            pl.BlockSpec((bn, d_in), lambda i: (i, 0)),
            pl.BlockSpec((bn, k, d_in), lambda i: (i, 0, 0)),
            pl.BlockSpec((d_in, d_out), lambda i: (0, 0)),
            pl.BlockSpec((d_in, d_out), lambda i: (0, 0)),
        ],
        out_specs=pl.BlockSpec((bn, d_out), lambda i: (i, 0)),
        out_shape=jax.ShapeDtypeStruct((n, d_out), jnp.float32),
    )(x, neigh_x, wl_t, wr_t)


# speedup vs baseline: 1.2221x; 1.2221x over previous
import jax
import jax.numpy as jnp
from jax.experimental import pallas as pl


def _body(x_ref, nx_ref, wl_ref, wr_ref, o_ref):
    k = nx_ref.shape[1]
    neigh = jnp.sum(nx_ref[...], axis=1) * (1.0 / k)
    o_ref[...] = (
        jnp.dot(x_ref[...], wl_ref[...], preferred_element_type=jnp.float32)
        + jnp.dot(neigh, wr_ref[...], preferred_element_type=jnp.float32)
    )


def kernel(x, neigh_x, W_l, W_r):
    n, d_in = x.shape
    _, k, _ = neigh_x.shape
    d_out = W_l.shape[0]
    bn = 400
    assert n % bn == 0
    wl_t = W_l.T
    wr_t = W_r.T
    return pl.pallas_call(
        _body,
        grid=(n // bn,),
        in_specs=[
            pl.BlockSpec((bn, d_in), lambda i: (i, 0)),
            pl.BlockSpec((bn, k, d_in), lambda i: (i, 0, 0)),
            pl.BlockSpec((d_in, d_out), lambda i: (0, 0)),
            pl.BlockSpec((d_in, d_out), lambda i: (0, 0)),
        ],
        out_specs=pl.BlockSpec((bn, d_out), lambda i: (i, 0)),
        out_shape=jax.ShapeDtypeStruct((n, d_out), jnp.float32),
    )(x, neigh_x, wl_t, wr_t)


# speedup vs baseline: 1.2941x; 1.0590x over previous
import jax
import jax.numpy as jnp
from jax.experimental import pallas as pl

_DN = (((1,), (1,)), ((), ()))


def _body(x_ref, nx_ref, wl_ref, wr_ref, o_ref):
    k = nx_ref.shape[1]
    neigh = jnp.sum(nx_ref[...], axis=1) * (1.0 / k)
    o_ref[...] = (
        jax.lax.dot_general(x_ref[...], wl_ref[...], _DN,
                            preferred_element_type=jnp.float32)
        + jax.lax.dot_general(neigh, wr_ref[...], _DN,
                              preferred_element_type=jnp.float32)
    )


def kernel(x, neigh_x, W_l, W_r):
    n, d_in = x.shape
    _, k, _ = neigh_x.shape
    d_out = W_l.shape[0]
    bn = 400
    assert n % bn == 0
    return pl.pallas_call(
        _body,
        grid=(n // bn,),
        in_specs=[
            pl.BlockSpec((bn, d_in), lambda i: (i, 0)),
            pl.BlockSpec((bn, k, d_in), lambda i: (i, 0, 0)),
            pl.BlockSpec((d_out, d_in), lambda i: (0, 0)),
            pl.BlockSpec((d_out, d_in), lambda i: (0, 0)),
        ],
        out_specs=pl.BlockSpec((bn, d_out), lambda i: (i, 0)),
        out_shape=jax.ShapeDtypeStruct((n, d_out), jnp.float32),
    )(x, neigh_x, W_l, W_r)
